# Initial kernel scaffold; baseline (speedup 1.0000x reference)
import jax
import jax.numpy as jnp
from jax.experimental import pallas as pl


def kernel(x):
    raise NotImplementedError("write your pallas kernel here")


# speedup vs baseline: 6.9818x; 6.9818x over previous
import functools

import jax
import jax.numpy as jnp
import numpy as np
from jax.experimental import pallas as pl
from jax.experimental.pallas import tpu as pltpu

_CHUNK = 128


def _cumsum_kernel(block_w, x_ref, t_ref, o_ref, carry_ref):
    k = pl.program_id(0)

    @pl.when(k == 0)
    def _():
        carry_ref[...] = jnp.zeros_like(carry_ref)

    t = t_ref[...]
    carry = carry_ref[...]
    for c in range(block_w // _CHUNK):
        xb = x_ref[:, c * _CHUNK:(c + 1) * _CHUNK]
        local = jax.lax.dot(xb, t, preferred_element_type=jnp.float32)
        out = local + carry
        o_ref[:, c * _CHUNK:(c + 1) * _CHUNK] = out
        carry = out[:, _CHUNK - 1:_CHUNK]
    carry_ref[...] = carry


@jax.jit
def kernel(x):
    rows, n = x.shape
    block_w = 2048
    tri = jnp.asarray(np.triu(np.ones((_CHUNK, _CHUNK), np.float32)))
    return pl.pallas_call(
        functools.partial(_cumsum_kernel, block_w),
        grid=(n // block_w,),
        in_specs=[
            pl.BlockSpec((rows, block_w), lambda k: (0, k)),
            pl.BlockSpec((_CHUNK, _CHUNK), lambda k: (0, 0)),
        ],
        out_specs=pl.BlockSpec((rows, block_w), lambda k: (0, k)),
        out_shape=jax.ShapeDtypeStruct((rows, n), jnp.float32),
        scratch_shapes=[pltpu.VMEM((rows, 1), jnp.float32)],
    )(x, tri)


# speedup vs baseline: 8.1068x; 1.1611x over previous
import functools

import jax
import jax.numpy as jnp
import numpy as np
from jax.experimental import pallas as pl
from jax.experimental.pallas import tpu as pltpu

_CHUNK = 128


def _cumsum_kernel(block_w, x_ref, t_ref, o_ref, carry_ref):
    k = pl.program_id(0)

    @pl.when(k == 0)
    def _():
        carry_ref[...] = jnp.zeros_like(carry_ref)

    t = t_ref[...]
    carry = carry_ref[...]
    for c in range(block_w // _CHUNK):
        xb = x_ref[:, c * _CHUNK:(c + 1) * _CHUNK]
        local = jax.lax.dot(xb, t, preferred_element_type=jnp.float32)
        out = local + carry
        o_ref[:, c * _CHUNK:(c + 1) * _CHUNK] = out
        carry = out[:, _CHUNK - 1:_CHUNK]
    carry_ref[...] = carry


@jax.jit
def kernel(x):
    rows, n = x.shape
    block_w = 4096
    tri = jnp.asarray(np.triu(np.ones((_CHUNK, _CHUNK), np.float32)))
    return pl.pallas_call(
        functools.partial(_cumsum_kernel, block_w),
        grid=(n // block_w,),
        in_specs=[
            pl.BlockSpec((rows, block_w), lambda k: (0, k)),
            pl.BlockSpec((_CHUNK, _CHUNK), lambda k: (0, 0)),
        ],
        out_specs=pl.BlockSpec((rows, block_w), lambda k: (0, k)),
        out_shape=jax.ShapeDtypeStruct((rows, n), jnp.float32),
        scratch_shapes=[pltpu.VMEM((rows, 1), jnp.float32)],
    )(x, tri)
